# baseline (device time: 52025 ns/iter reference)
import jax
import jax.numpy as jnp
from jax import lax
from jax.experimental import pallas as pl
from jax.experimental.pallas import tpu as pltpu

N_DEV = 4


def kernel(x, Win0, Wout0, Win1, Wout1, Win2, Wout2):
    m_per, d = x.shape
    M = N_DEV * m_per

    def body(x_ref, win0_ref, wout0_ref, win1_ref, wout1_ref,
             win2_ref, wout2_ref, out_ref,
             xg_ref, part_ref, rs_ref,
             ag_send_sems, ag_recv_sems, rs_send_sems, rs_recv_sems):
        me = lax.axis_index("i")

        barrier_sem = pltpu.get_barrier_semaphore()
        for off in (1, 2, 3):
            p = (me + off) % N_DEV
            pl.semaphore_signal(barrier_sem, inc=1, device_id=(p,),
                                device_id_type=pl.DeviceIdType.MESH)
        pl.semaphore_wait(barrier_sem, N_DEV - 1)

        def rows(pos):
            return pl.ds(pos * m_per, m_per)

        def all_gather(local_bf16):
            xg_ref[rows(me), :] = local_bf16
            sends = []
            for off in (1, 2, 3):
                p = (me + off) % N_DEV
                s = pltpu.make_async_remote_copy(
                    src_ref=xg_ref.at[rows(me), :],
                    dst_ref=xg_ref.at[rows(me), :],
                    send_sem=ag_send_sems.at[off - 1],
                    recv_sem=ag_recv_sems.at[off - 1],
                    device_id=(p,),
                    device_id_type=pl.DeviceIdType.MESH,
                )
                s.start()
                sends.append(s)
            for off_r in (1, 2, 3):
                q = (me + off_r) % N_DEV
                r = pltpu.make_async_remote_copy(
                    src_ref=xg_ref.at[rows(me), :],
                    dst_ref=xg_ref.at[rows(q), :],
                    send_sem=ag_send_sems.at[0],
                    recv_sem=ag_recv_sems.at[3 - off_r],
                    device_id=(q,),
                    device_id_type=pl.DeviceIdType.MESH,
                )
                r.wait_recv()
            for s in sends:
                s.wait_send()
            return xg_ref[:, :]

        def reduce_scatter(partial_f32):
            part_ref[:, :] = partial_f32
            rs_ref[rows(me), :] = part_ref[rows(me), :]
            sends = []
            for off in (1, 2, 3):
                p = (me + off) % N_DEV
                s = pltpu.make_async_remote_copy(
                    src_ref=part_ref.at[rows(p), :],
                    dst_ref=rs_ref.at[rows(me), :],
                    send_sem=rs_send_sems.at[off - 1],
                    recv_sem=rs_recv_sems.at[off - 1],
                    device_id=(p,),
                    device_id_type=pl.DeviceIdType.MESH,
                )
                s.start()
                sends.append(s)
            for off_r in (1, 2, 3):
                q = (me + off_r) % N_DEV
                r = pltpu.make_async_remote_copy(
                    src_ref=part_ref.at[rows(me), :],
                    dst_ref=rs_ref.at[rows(q), :],
                    send_sem=rs_send_sems.at[0],
                    recv_sem=rs_recv_sems.at[3 - off_r],
                    device_id=(q,),
                    device_id_type=pl.DeviceIdType.MESH,
                )
                r.wait_recv()
            for s in sends:
                s.wait_send()
            chunks = rs_ref[:, :].reshape(N_DEV, m_per, d)
            return jnp.sum(chunks, axis=0)

        layers = [(win0_ref, wout0_ref), (win1_ref, wout1_ref),
                  (win2_ref, wout2_ref)]
        xb = x_ref[:, :].astype(jnp.bfloat16)
        for l, (win_ref, wout_ref) in enumerate(layers):
            xg = all_gather(xb)
            w_in = win_ref[:, :].astype(jnp.bfloat16)
            w_out = wout_ref[:, :].astype(jnp.bfloat16)
            h = jnp.maximum(
                jnp.dot(xg, w_in, preferred_element_type=jnp.float32), 0.0)
            partial = jnp.dot(h.astype(jnp.bfloat16), w_out,
                              preferred_element_type=jnp.float32)
            red = reduce_scatter(partial)
            if l < len(layers) - 1:
                xb = red.astype(jnp.bfloat16)
            else:
                out_ref[:, :] = red

    return pl.pallas_call(
        body,
        out_shape=jax.ShapeDtypeStruct((m_per, d), jnp.float32),
        in_specs=[pl.BlockSpec(memory_space=pltpu.VMEM)] * 7,
        out_specs=pl.BlockSpec(memory_space=pltpu.VMEM),
        scratch_shapes=[
            pltpu.VMEM((M, d), jnp.bfloat16),
            pltpu.VMEM((M, d), jnp.float32),
            pltpu.VMEM((M, d), jnp.float32),
            pltpu.SemaphoreType.DMA((3,)),
            pltpu.SemaphoreType.DMA((3,)),
            pltpu.SemaphoreType.DMA((3,)),
            pltpu.SemaphoreType.DMA((3,)),
        ],
        compiler_params=pltpu.CompilerParams(collective_id=0),
    )(x, Win0, Wout0, Win1, Wout1, Win2, Wout2)


# device time: 39328 ns/iter; 1.3228x vs baseline; 1.3228x over previous
import jax
import jax.numpy as jnp
from jax import lax
from jax.experimental import pallas as pl
from jax.experimental.pallas import tpu as pltpu

N_DEV = 4
_OFFS = (1, 3, 2)


def kernel(x, Win0, Wout0, Win1, Wout1, Win2, Wout2):
    m_per, d = x.shape
    M = N_DEV * m_per

    def body(x_ref, win0_ref, wout0_ref, win1_ref, wout1_ref,
             win2_ref, wout2_ref, out_ref,
             xg_ref, part_ref, rs_ref,
             ag_send_sems, ag_recv_sems, rs_send_sems, rs_recv_sems):
        me = lax.axis_index("i")

        barrier_sem = pltpu.get_barrier_semaphore()
        for off in _OFFS:
            p = (me + off) % N_DEV
            pl.semaphore_signal(barrier_sem, inc=1, device_id=(p,),
                                device_id_type=pl.DeviceIdType.MESH)
        pl.semaphore_wait(barrier_sem, N_DEV - 1)

        def rows(pos):
            return pl.ds(pos * m_per, m_per)

        def layer(xb, win_ref, wout_ref, is_last):
            xg_ref[rows(me), :] = xb
            ag_sends = []
            for off in _OFFS:
                p = (me + off) % N_DEV
                s = pltpu.make_async_remote_copy(
                    src_ref=xg_ref.at[rows(me), :],
                    dst_ref=xg_ref.at[rows(me), :],
                    send_sem=ag_send_sems.at[off - 1],
                    recv_sem=ag_recv_sems.at[off - 1],
                    device_id=(p,),
                    device_id_type=pl.DeviceIdType.MESH,
                )
                s.start()
                ag_sends.append(s)

            w_in = win_ref[:, :].astype(jnp.bfloat16)
            w_out = wout_ref[:, :].astype(jnp.bfloat16)

            def chunk_partial(xc):
                h = jnp.maximum(
                    jnp.dot(xc, w_in, preferred_element_type=jnp.float32),
                    0.0)
                return jnp.dot(h.astype(jnp.bfloat16), w_out,
                               preferred_element_type=jnp.float32)

            acc = chunk_partial(xb)

            rs_sends = []
            for off_r in _OFFS:
                q = (me + off_r) % N_DEV
                r = pltpu.make_async_remote_copy(
                    src_ref=xg_ref.at[rows(me), :],
                    dst_ref=xg_ref.at[rows(q), :],
                    send_sem=ag_send_sems.at[0],
                    recv_sem=ag_recv_sems.at[3 - off_r],
                    device_id=(q,),
                    device_id_type=pl.DeviceIdType.MESH,
                )
                r.wait_recv()
                part_q = chunk_partial(xg_ref[rows(q), :])
                part_ref[rows(q), :] = part_q.astype(jnp.bfloat16)
                s = pltpu.make_async_remote_copy(
                    src_ref=part_ref.at[rows(q), :],
                    dst_ref=rs_ref.at[rows(me), :],
                    send_sem=rs_send_sems.at[off_r - 1],
                    recv_sem=rs_recv_sems.at[off_r - 1],
                    device_id=(q,),
                    device_id_type=pl.DeviceIdType.MESH,
                )
                s.start()
                rs_sends.append(s)

            for off_r in _OFFS:
                q = (me + off_r) % N_DEV
                r = pltpu.make_async_remote_copy(
                    src_ref=part_ref.at[rows(me), :],
                    dst_ref=rs_ref.at[rows(q), :],
                    send_sem=rs_send_sems.at[0],
                    recv_sem=rs_recv_sems.at[3 - off_r],
                    device_id=(q,),
                    device_id_type=pl.DeviceIdType.MESH,
                )
                r.wait_recv()
                acc = acc + rs_ref[rows(q), :].astype(jnp.float32)

            for s in ag_sends:
                s.wait_send()
            for s in rs_sends:
                s.wait_send()

            if is_last:
                out_ref[:, :] = acc
                return None
            return acc.astype(jnp.bfloat16)

        xb = x_ref[:, :].astype(jnp.bfloat16)
        xb = layer(xb, win0_ref, wout0_ref, False)
        xb = layer(xb, win1_ref, wout1_ref, False)
        layer(xb, win2_ref, wout2_ref, True)

    return pl.pallas_call(
        body,
        out_shape=jax.ShapeDtypeStruct((m_per, d), jnp.float32),
        in_specs=[pl.BlockSpec(memory_space=pltpu.VMEM)] * 7,
        out_specs=pl.BlockSpec(memory_space=pltpu.VMEM),
        scratch_shapes=[
            pltpu.VMEM((M, d), jnp.bfloat16),
            pltpu.VMEM((M, d), jnp.bfloat16),
            pltpu.VMEM((M, d), jnp.bfloat16),
            pltpu.SemaphoreType.DMA((3,)),
            pltpu.SemaphoreType.DMA((3,)),
            pltpu.SemaphoreType.DMA((3,)),
            pltpu.SemaphoreType.DMA((3,)),
        ],
        compiler_params=pltpu.CompilerParams(collective_id=0),
    )(x, Win0, Wout0, Win1, Wout1, Win2, Wout2)
